# Initial kernel scaffold; baseline (speedup 1.0000x reference)
import jax
import jax.numpy as jnp
from jax.experimental import pallas as pl


def kernel(words, table):
    raise NotImplementedError("write your pallas kernel here")


# speedup vs baseline: 3.3438x; 3.3438x over previous
import functools

import jax
import jax.numpy as jnp
from jax import lax
from jax.experimental import pallas as pl
from jax.experimental.pallas import tpu as pltpu
from jax.experimental.pallas import tpu_sc as plsc

NC = 2
NS = 16
NW = NC * NS
CHUNK = 128
D = 128


@functools.cache
def _emb_kernel(n_idx: int):
  b_per_w = n_idx // NW
  n_chunks = b_per_w // CHUNK
  mesh = plsc.VectorSubcoreMesh(
      core_axis_name="c", subcore_axis_name="s", num_cores=NC, num_subcores=NS
  )

  @functools.partial(
      pl.kernel,
      out_type=jax.ShapeDtypeStruct((n_idx, D), jnp.float32),
      mesh=mesh,
      scratch_types=[
          pltpu.VMEM((b_per_w,), jnp.int32),
          pltpu.VMEM((2, CHUNK, D), jnp.float32),
          pltpu.SemaphoreType.DMA,
          pltpu.SemaphoreType.DMA,
      ],
  )
  def k(words_hbm, table_hbm, out_hbm, idx_v, rows_v, gsem, osem):
    wid = lax.axis_index("s") * NC + lax.axis_index("c")
    base = wid * b_per_w
    pltpu.sync_copy(words_hbm.at[pl.ds(base, b_per_w)], idx_v)

    pltpu.async_copy(
        table_hbm.at[idx_v.at[pl.ds(0, CHUNK)]], rows_v.at[0], gsem
    )

    def pair_body(i, _):
      for b in range(2):
        c = 2 * i + b
        nxt = c + 1
        @pl.when(nxt < n_chunks)
        def _():
          pltpu.async_copy(
              table_hbm.at[idx_v.at[pl.ds(nxt * CHUNK, CHUNK)]],
              rows_v.at[1 - b],
              gsem,
          )

        pltpu.make_async_copy(
            table_hbm.at[pl.ds(0, CHUNK)], rows_v.at[b], gsem
        ).wait()
        cp = pltpu.async_copy(
            rows_v.at[b], out_hbm.at[pl.ds(base + c * CHUNK, CHUNK)], osem
        )
        cp.wait()
      return 0

    lax.fori_loop(0, n_chunks // 2, pair_body, 0)

  return k


def kernel(words, table):
  b, h = words.shape
  idx = words.reshape(-1).astype(jnp.int32)
  out = _emb_kernel(idx.shape[0])(idx, table.astype(jnp.float32))
  return out.reshape(b, h, D)
